# baseline (speedup 1.0000x reference)
import functools

import jax
import jax.numpy as jnp
from jax import lax
from jax.experimental import pallas as pl
from jax.experimental.pallas import tpu as pltpu
from jax.experimental.pallas import tpu_sc as plsc

_N = 4194304
_M = _N // 2
_TC_BLOCK = 262144

_NW = 32
_SPAN = (_N - _M) // _NW
_C = 16384
_NCH = _SPAN // _C


def _tc_body(o_ref, t_ref, m_ref, r_ref):
    d = o_ref[...] - t_ref[...]
    r_ref[...] = jnp.where(m_ref[...], d * d, 0.0)


def _sc_body(o_hbm, t_hbm, m_hbm, out_hbm,
             o_v, t_v, m_v, r_v, semo, semt, semm, semr):
    wid = lax.axis_index("s") * 2 + lax.axis_index("c")
    base = _M + wid * _SPAN
    mbase = wid * _SPAN

    def in_copies(slot, ci):
        off = pl.multiple_of(base + ci * _C, _C)
        moff = pl.multiple_of(mbase + ci * _C, _C)
        return (
            pltpu.make_async_copy(
                o_hbm.at[pl.ds(off, _C)], o_v.at[slot], semo.at[slot]),
            pltpu.make_async_copy(
                t_hbm.at[pl.ds(off, _C)], t_v.at[slot], semt.at[slot]),
            pltpu.make_async_copy(
                m_hbm.at[pl.ds(moff, _C)], m_v.at[slot], semm.at[slot]),
        )

    def out_copy(slot, ci):
        moff = pl.multiple_of(mbase + ci * _C, _C)
        return pltpu.make_async_copy(
            r_v.at[slot], out_hbm.at[pl.ds(moff, _C)], semr.at[slot])

    for c in in_copies(0, 0):
        c.start()

    for ci in range(_NCH):
        slot = ci % 2
        if ci + 1 < _NCH:
            for c in in_copies(1 - slot, ci + 1):
                c.start()
        for c in in_copies(slot, ci):
            c.wait()
        if ci >= 2:
            out_copy(slot, ci - 2).wait()

        ov, tv, mv, rv = o_v.at[slot], t_v.at[slot], m_v.at[slot], r_v.at[slot]

        @plsc.parallel_loop(0, _C, step=16, unroll=8)
        def _(eb):
            ix = pl.multiple_of(eb, 16)
            o = ov[pl.ds(ix, 16)]
            t = tv[pl.ds(ix, 16)]
            m = mv[pl.ds(ix, 16)]
            d = o - t
            rv[pl.ds(ix, 16)] = jnp.where(m != 0, d * d, 0.0)

        out_copy(slot, ci).start()

    out_copy(_NCH % 2, _NCH - 2).wait()
    out_copy(1 - _NCH % 2, _NCH - 1).wait()


def kernel(outputs, targets, precondition):
    m1 = precondition.reshape(_N)

    mesh = plsc.VectorSubcoreMesh(core_axis_name="c", subcore_axis_name="s")
    sc_run = functools.partial(
        pl.kernel,
        mesh=mesh,
        out_type=jax.ShapeDtypeStruct((_N - _M,), jnp.float32),
        scratch_types=[
            pltpu.VMEM((2, _C), jnp.float32),
            pltpu.VMEM((2, _C), jnp.float32),
            pltpu.VMEM((2, _C), jnp.int32),
            pltpu.VMEM((2, _C), jnp.float32),
            pltpu.SemaphoreType.DMA((2,)),
            pltpu.SemaphoreType.DMA((2,)),
            pltpu.SemaphoreType.DMA((2,)),
            pltpu.SemaphoreType.DMA((2,)),
        ],
    )(_sc_body)
    sc_out = sc_run(outputs, targets, m1[_M:])

    spec = pl.BlockSpec((_TC_BLOCK,), lambda i: (i,))
    tc_out = pl.pallas_call(
        _tc_body,
        grid=(_M // _TC_BLOCK,),
        in_specs=[spec, spec, spec],
        out_specs=spec,
        out_shape=jax.ShapeDtypeStruct((_M,), jnp.float32),
    )(outputs, targets, m1)

    return jnp.concatenate([tc_out, sc_out]).reshape(_N, 1)


# speedup vs baseline: 2.7437x; 2.7437x over previous
import functools

import jax
import jax.numpy as jnp
from jax import lax
from jax.experimental import pallas as pl
from jax.experimental.pallas import tpu as pltpu
from jax.experimental.pallas import tpu_sc as plsc

_N = 4194304
_NW = 32
_SPAN = _N // _NW
_C = 16384
_NCH = _SPAN // _C

_PACK_BLOCK = 524288

_GATHER_DNUMS = lax.GatherDimensionNumbers(
    offset_dims=(), collapsed_slice_dims=(0,), start_index_map=(0,))


def _vgather(vec, idx):
    return lax.gather(vec, idx[:, None], _GATHER_DNUMS, slice_sizes=(1,),
                      mode=lax.GatherScatterMode.PROMISE_IN_BOUNDS)


def _cast_body(m_ref, w_ref):
    w_ref[...] = m_ref[...].astype(jnp.int32)


def _sc_body(o_hbm, t_hbm, m_hbm, out_hbm,
             o_v, t_v, m_v, r_v, semo, semt, semm, semr):
    wid = lax.axis_index("s") * 2 + lax.axis_index("c")
    base = wid * _SPAN

    lane = lax.iota(jnp.int32, 16)
    word_idx = lane >> 2
    shifts = (lane & 3) << 3

    def in_copies(slot, ci):
        off = pl.multiple_of(base + ci * _C, _C)
        return (
            pltpu.make_async_copy(
                o_hbm.at[pl.ds(off, _C)], o_v.at[slot], semo.at[slot]),
            pltpu.make_async_copy(
                t_hbm.at[pl.ds(off, _C)], t_v.at[slot], semt.at[slot]),
            pltpu.make_async_copy(
                m_hbm.at[pl.ds(off, _C)], m_v.at[slot], semm.at[slot]),
        )

    def out_copy(slot, ci):
        off = pl.multiple_of(base + ci * _C, _C)
        return pltpu.make_async_copy(
            r_v.at[slot], out_hbm.at[pl.ds(off, _C)], semr.at[slot])

    for c in in_copies(0, 0):
        c.start()

    for ci in range(_NCH):
        slot = ci % 2
        if ci + 1 < _NCH:
            for c in in_copies(1 - slot, ci + 1):
                c.start()
        for c in in_copies(slot, ci):
            c.wait()
        if ci >= 2:
            out_copy(slot, ci - 2).wait()

        ov, tv, mv, rv = o_v.at[slot], t_v.at[slot], m_v.at[slot], r_v.at[slot]

        @plsc.parallel_loop(0, _C, step=16, unroll=8)
        def _(eb):
            ix = pl.multiple_of(eb, 16)
            o = ov[pl.ds(ix, 16)]
            t = tv[pl.ds(ix, 16)]
            m = mv[pl.ds(ix, 16)]
            d = o - t
            rv[pl.ds(ix, 16)] = jnp.where(m != 0, d * d, 0.0)

        out_copy(slot, ci).start()

    out_copy(_NCH % 2, _NCH - 2).wait()
    out_copy(1 - _NCH % 2, _NCH - 1).wait()


def kernel(outputs, targets, precondition):
    m1 = precondition.reshape(_N)
    mspec = pl.BlockSpec((_PACK_BLOCK,), lambda i: (i,))
    m32 = pl.pallas_call(
        _cast_body,
        grid=(_N // _PACK_BLOCK,),
        in_specs=[mspec],
        out_specs=mspec,
        out_shape=jax.ShapeDtypeStruct((_N,), jnp.int32),
    )(m1)

    mesh = plsc.VectorSubcoreMesh(core_axis_name="c", subcore_axis_name="s")
    run = functools.partial(
        pl.kernel,
        mesh=mesh,
        out_type=jax.ShapeDtypeStruct((_N,), jnp.float32),
        scratch_types=[
            pltpu.VMEM((2, _C), jnp.float32),
            pltpu.VMEM((2, _C), jnp.float32),
            pltpu.VMEM((2, _C), jnp.int32),
            pltpu.VMEM((2, _C), jnp.float32),
            pltpu.SemaphoreType.DMA((2,)),
            pltpu.SemaphoreType.DMA((2,)),
            pltpu.SemaphoreType.DMA((2,)),
            pltpu.SemaphoreType.DMA((2,)),
        ],
    )(_sc_body)
    out = run(outputs, targets, m32)
    return out.reshape(_N, 1)


# speedup vs baseline: 3.2515x; 1.1850x over previous
import functools

import jax
import jax.numpy as jnp
from jax import lax
from jax.experimental import pallas as pl
from jax.experimental.pallas import tpu as pltpu
from jax.experimental.pallas import tpu_sc as plsc

_N = 4194304
_NW = 32
_SPAN = _N // _NW
_C = 16384
_NCH = _SPAN // _C


def _sc_body(o_hbm, t_hbm, m_hbm, out_hbm,
             o_v, t_v, m_v, r_v, semo, semt, semm, semr):
    wid = lax.axis_index("s") * 2 + lax.axis_index("c")
    base = wid * _SPAN

    def in_copies(slot, ci):
        off = pl.multiple_of(base + ci * _C, _C)
        return (
            pltpu.make_async_copy(
                o_hbm.at[pl.ds(off, _C)], o_v.at[slot], semo.at[slot]),
            pltpu.make_async_copy(
                t_hbm.at[pl.ds(off, _C)], t_v.at[slot], semt.at[slot]),
            pltpu.make_async_copy(
                m_hbm.at[pl.ds(off, _C)], m_v.at[slot], semm.at[slot]),
        )

    def out_copy(slot, ci):
        off = pl.multiple_of(base + ci * _C, _C)
        return pltpu.make_async_copy(
            r_v.at[slot], out_hbm.at[pl.ds(off, _C)], semr.at[slot])

    for c in in_copies(0, 0):
        c.start()

    for ci in range(_NCH):
        slot = ci % 2
        if ci + 1 < _NCH:
            for c in in_copies(1 - slot, ci + 1):
                c.start()
        for c in in_copies(slot, ci):
            c.wait()
        if ci >= 2:
            out_copy(slot, ci - 2).wait()

        ov, tv, mv, rv = o_v.at[slot], t_v.at[slot], m_v.at[slot], r_v.at[slot]

        @plsc.parallel_loop(0, _C, step=16, unroll=8)
        def _(eb):
            ix = pl.multiple_of(eb, 16)
            o = ov[pl.ds(ix, 16)]
            t = tv[pl.ds(ix, 16)]
            m = mv[pl.ds(ix, 16)]
            d = o - t
            rv[pl.ds(ix, 16)] = jnp.where(m != 0, d * d, 0.0)

        out_copy(slot, ci).start()

    out_copy(_NCH % 2, _NCH - 2).wait()
    out_copy(1 - _NCH % 2, _NCH - 1).wait()


def kernel(outputs, targets, precondition):
    m1 = precondition.reshape(_N)
    mesh = plsc.VectorSubcoreMesh(core_axis_name="c", subcore_axis_name="s")
    run = functools.partial(
        pl.kernel,
        mesh=mesh,
        out_type=jax.ShapeDtypeStruct((_N,), jnp.float32),
        scratch_types=[
            pltpu.VMEM((2, _C), jnp.float32),
            pltpu.VMEM((2, _C), jnp.float32),
            pltpu.VMEM((2, _C), jnp.int32),
            pltpu.VMEM((2, _C), jnp.float32),
            pltpu.SemaphoreType.DMA((2,)),
            pltpu.SemaphoreType.DMA((2,)),
            pltpu.SemaphoreType.DMA((2,)),
            pltpu.SemaphoreType.DMA((2,)),
        ],
    )(_sc_body)
    out = run(outputs, targets, m1)
    return out.reshape(_N, 1)
